# Initial kernel scaffold; baseline (speedup 1.0000x reference)
import jax
import jax.numpy as jnp
from jax.experimental import pallas as pl


def kernel(emb_weight, pos_rw, neg_rw):
    raise NotImplementedError("write your pallas kernel here")


# speedup vs baseline: 9.4303x; 9.4303x over previous
import functools

import jax
import jax.numpy as jnp
from jax import lax
from jax.experimental import pallas as pl
from jax.experimental.pallas import tpu as pltpu
from jax.experimental.pallas import tpu_sc as plsc

EPS = 1e-15
D = 128
B = 16384
R = 20
NC, NS, L = 2, 16, 16
NW = NC * NS
WALKS_PER_WORKER = 2 * B // NW
CHUNK = 32
N_CHUNKS = WALKS_PER_WORKER // CHUNK
IDX_PER_CHUNK = CHUNK * R
GATHER_N = 128
N_GATHERS = IDX_PER_CHUNK // GATHER_N


def _sc_dots_body(emb_hbm, starts_hbm, rests_hbm, dots_hbm,
                  sidx, ridx, srow, rrow, dout, sem):
    wid = lax.axis_index("s") * NC + lax.axis_index("c")
    walk_base = wid * WALKS_PER_WORKER

    def chunk_body(ci, carry):
        wbase = walk_base + ci * CHUNK
        rbase = wbase * R
        pltpu.sync_copy(starts_hbm.at[pl.ds(wbase, CHUNK)], sidx)
        pltpu.sync_copy(rests_hbm.at[pl.ds(rbase, IDX_PER_CHUNK)], ridx)
        cp_s = pltpu.async_copy(emb_hbm.at[sidx], srow, sem)
        cps = [
            pltpu.async_copy(
                emb_hbm.at[ridx.at[pl.ds(g * GATHER_N, GATHER_N)]],
                rrow.at[pl.ds(g * GATHER_N, GATHER_N)], sem)
            for g in range(N_GATHERS)
        ]
        cp_s.wait()
        for cp in cps:
            cp.wait()

        lane = lax.iota(jnp.int32, L)

        def walk_body(w, c2):
            svec = [srow[w, pl.ds(k * L, L)] for k in range(D // L)]
            dv0 = jnp.zeros((L,), jnp.float32)
            dv1 = jnp.zeros((L,), jnp.float32)
            for j in range(R):
                r = w * R + j
                acc = svec[0] * rrow[r, pl.ds(0, L)]
                for k in range(1, D // L):
                    acc = acc + svec[k] * rrow[r, pl.ds(k * L, L)]
                for sh in (8, 4, 2, 1):
                    acc = acc + acc.at[lane ^ sh].get(
                        mode="promise_in_bounds")
                if j < L:
                    dv0 = jnp.where(lane == j, acc, dv0)
                else:
                    dv1 = jnp.where(lane == (j - L), acc, dv1)
            dout[pl.ds(w * R, L)] = dv0
            dout[pl.ds(w * R + L, L)] = dv1
            return c2

        lax.fori_loop(0, CHUNK, walk_body, 0, unroll=False)
        pltpu.sync_copy(dout.at[pl.ds(0, IDX_PER_CHUNK)],
                        dots_hbm.at[pl.ds(rbase, IDX_PER_CHUNK)])
        return carry

    lax.fori_loop(0, N_CHUNKS, chunk_body, 0, unroll=False)


_sc_dots = functools.partial(
    pl.kernel,
    out_type=jax.ShapeDtypeStruct((2 * B * R,), jnp.float32),
    mesh=plsc.VectorSubcoreMesh(core_axis_name="c", subcore_axis_name="s"),
    scratch_types=[
        pltpu.VMEM((CHUNK,), jnp.int32),
        pltpu.VMEM((IDX_PER_CHUNK,), jnp.int32),
        pltpu.VMEM((CHUNK, D), jnp.float32),
        pltpu.VMEM((IDX_PER_CHUNK, D), jnp.float32),
        pltpu.VMEM((IDX_PER_CHUNK + L,), jnp.float32),
        pltpu.SemaphoreType.DMA,
    ],
)(_sc_dots_body)


def _tc_loss_body(dots_ref, out_ref):
    x = dots_ref[...]
    half = x.shape[0] // 2
    pos = x[:half]
    neg = x[half:]
    pos_terms = -jnp.log(jax.nn.sigmoid(pos) + EPS)
    neg_terms = -jnp.log(1.0 - jax.nn.sigmoid(neg) + EPS)
    out_ref[0, 0] = (jnp.sum(pos_terms) + jnp.sum(neg_terms)) / (B * R)


_tc_loss = pl.pallas_call(
    _tc_loss_body,
    out_shape=jax.ShapeDtypeStruct((1, 1), jnp.float32),
    in_specs=[pl.BlockSpec(memory_space=pltpu.VMEM)],
    out_specs=pl.BlockSpec(memory_space=pltpu.SMEM),
)


def kernel(emb_weight, pos_rw, neg_rw):
    starts = jnp.concatenate([pos_rw[:, 0], neg_rw[:, 0]])
    rests = jnp.concatenate(
        [pos_rw[:, 1:].reshape(-1), neg_rw[:, 1:].reshape(-1)])
    dots = _sc_dots(emb_weight, starts, rests)
    loss = _tc_loss(dots.reshape(2 * B * R // 256, 256))
    return loss[0, 0]


# speedup vs baseline: 14.7278x; 1.5618x over previous
import functools

import jax
import jax.numpy as jnp
from jax import lax
from jax.experimental import pallas as pl
from jax.experimental.pallas import tpu as pltpu
from jax.experimental.pallas import tpu_sc as plsc

EPS = 1e-15
D = 128
B = 16384
W = 21
R = 20
NC, NS, L = 2, 16, 16
NW = NC * NS
WALKS_PER_WORKER = 2 * B // NW
CHUNK = 16
N_CHUNKS = WALKS_PER_WORKER // CHUNK
ROWS_PER_CHUNK = CHUNK * W
DOTS_PER_CHUNK = CHUNK * R


def _sc_dots_body(emb_hbm, walk_idx_hbm, dots_hbm,
                  idx0, idx1, rows0, rows1, dout0, dout1,
                  gsem0, gsem1, ssem0, ssem1):
    wid = lax.axis_index("s") * NC + lax.axis_index("c")
    walk_base = wid * WALKS_PER_WORKER
    idxs, rows, douts = (idx0, idx1), (rows0, rows1), (dout0, dout1)
    gsems, ssems = (gsem0, gsem1), (ssem0, ssem1)

    def fire(ci, b):
        pltpu.sync_copy(
            walk_idx_hbm.at[pl.ds((walk_base + ci * CHUNK) * W,
                                  ROWS_PER_CHUNK)],
            idxs[b])
        pltpu.async_copy(emb_hbm.at[idxs[b]], rows[b], gsems[b])

    def drain_gather(b):
        pltpu.make_async_copy(emb_hbm.at[idxs[b]], rows[b], gsems[b]).wait()

    def store(ci, b):
        pltpu.async_copy(
            douts[b].at[pl.ds(0, DOTS_PER_CHUNK)],
            dots_hbm.at[pl.ds((walk_base + ci * CHUNK) * R,
                              DOTS_PER_CHUNK)],
            ssems[b])

    def drain_store(ci, b):
        pltpu.make_async_copy(
            douts[b].at[pl.ds(0, DOTS_PER_CHUNK)],
            dots_hbm.at[pl.ds((walk_base + ci * CHUNK) * R,
                              DOTS_PER_CHUNK)],
            ssems[b]).wait()

    lane = lax.iota(jnp.int32, L)

    def compute(b):
        rbuf, dbuf = rows[b], douts[b]

        def walk_body(w, c2):
            base = w * W
            svec = [rbuf[base, pl.ds(k * L, L)] for k in range(D // L)]
            dv0 = jnp.zeros((L,), jnp.float32)
            dv1 = jnp.zeros((L,), jnp.float32)
            for j in range(R):
                r = base + 1 + j
                acc = svec[0] * rbuf[r, pl.ds(0, L)]
                for k in range(1, D // L):
                    acc = acc + svec[k] * rbuf[r, pl.ds(k * L, L)]
                for sh in (8, 4, 2, 1):
                    acc = acc + acc.at[lane ^ sh].get(
                        mode="promise_in_bounds")
                if j < L:
                    dv0 = jnp.where(lane == j, acc, dv0)
                else:
                    dv1 = jnp.where(lane == (j - L), acc, dv1)
            dbuf[pl.ds(w * R, L)] = dv0
            dbuf[pl.ds(w * R + L, L)] = dv1
            return c2

        lax.fori_loop(0, CHUNK, walk_body, 0, unroll=False)

    fire(0, 0)

    def outer(g, carry):
        for b in range(2):
            ci = g * 2 + b

            @pl.when(ci + 1 < N_CHUNKS)
            def _():
                fire(ci + 1, 1 - b)

            drain_gather(b)

            @pl.when(ci >= 2)
            def _():
                drain_store(ci, b)

            compute(b)
            store(ci, b)
        return carry

    lax.fori_loop(0, N_CHUNKS // 2, outer, 0, unroll=False)
    drain_store(N_CHUNKS - 2, 0)
    drain_store(N_CHUNKS - 1, 1)


_sc_dots = functools.partial(
    pl.kernel,
    out_type=jax.ShapeDtypeStruct((2 * B * R,), jnp.float32),
    mesh=plsc.VectorSubcoreMesh(core_axis_name="c", subcore_axis_name="s"),
    scratch_types=[
        pltpu.VMEM((ROWS_PER_CHUNK,), jnp.int32),
        pltpu.VMEM((ROWS_PER_CHUNK,), jnp.int32),
        pltpu.VMEM((ROWS_PER_CHUNK, D), jnp.float32),
        pltpu.VMEM((ROWS_PER_CHUNK, D), jnp.float32),
        pltpu.VMEM((DOTS_PER_CHUNK + L,), jnp.float32),
        pltpu.VMEM((DOTS_PER_CHUNK + L,), jnp.float32),
        pltpu.SemaphoreType.DMA,
        pltpu.SemaphoreType.DMA,
        pltpu.SemaphoreType.DMA,
        pltpu.SemaphoreType.DMA,
    ],
)(_sc_dots_body)


def _tc_loss_body(dots_ref, out_ref):
    x = dots_ref[...]
    half = x.shape[0] // 2
    pos = x[:half]
    neg = x[half:]
    pos_terms = -jnp.log(jax.nn.sigmoid(pos) + EPS)
    neg_terms = -jnp.log(1.0 - jax.nn.sigmoid(neg) + EPS)
    out_ref[0, 0] = (jnp.sum(pos_terms) + jnp.sum(neg_terms)) / (B * R)


_tc_loss = pl.pallas_call(
    _tc_loss_body,
    out_shape=jax.ShapeDtypeStruct((1, 1), jnp.float32),
    in_specs=[pl.BlockSpec(memory_space=pltpu.VMEM)],
    out_specs=pl.BlockSpec(memory_space=pltpu.SMEM),
)


def kernel(emb_weight, pos_rw, neg_rw):
    walk_idx = jnp.concatenate([pos_rw.reshape(-1), neg_rw.reshape(-1)])
    dots = _sc_dots(emb_weight, walk_idx)
    loss = _tc_loss(dots.reshape(2 * B * R // 256, 256))
    return loss[0, 0]
